# Initial kernel scaffold; baseline (speedup 1.0000x reference)
import jax
import jax.numpy as jnp
from jax.experimental import pallas as pl


def kernel(x, edge_index, edge_vals, poly_logits, hp_alpha):
    raise NotImplementedError("write your pallas kernel here")


# speedup vs baseline: 2.3964x; 2.3964x over previous
import functools

import jax
import jax.numpy as jnp
from jax import lax
from jax.experimental import pallas as pl
from jax.experimental.pallas import tpu as pltpu
from jax.experimental.pallas import tpu_sc as plsc

G = 128
RC = 32
NTILE = 16
NCORE = 2


def _degmax_tc(a, b):
  def body(a_ref, b_ref, o_ref):
    m = jnp.max(b_ref[...] - a_ref[...])
    o_ref[...] = jnp.full((8, 128), m, jnp.float32)
  return pl.pallas_call(
      body, out_shape=jax.ShapeDtypeStruct((8, 128), jnp.float32))(a, b)


@functools.lru_cache(maxsize=None)
def _make_spmm(N, H, Ep, aux):
  HALF = N // 2
  PT = (-(-HALF // NTILE) + 7) // 8 * 8
  NQ = H // 16
  mesh = plsc.VectorSubcoreMesh(core_axis_name="c", subcore_axis_name="s")

  scratch = [
      pltpu.VMEM_SHARED((HALF, H), jnp.float32),
      pltpu.VMEM((G,), jnp.int32),
      pltpu.VMEM((G,), jnp.float32),
      pltpu.VMEM((G,), jnp.int32),
      pltpu.VMEM((G, H), jnp.float32),
      pltpu.VMEM((RC, H), jnp.float32),
      pltpu.VMEM((RC, H), jnp.float32),
      pltpu.VMEM((RC, H), jnp.float32),
      pltpu.VMEM((RC, H), jnp.float32),
      pltpu.VMEM((RC, H), jnp.float32),
      pltpu.VMEM((RC, H), jnp.float32),
      pltpu.VMEM((16,), jnp.float32),
      pltpu.VMEM((16,), jnp.int32),
      pltpu.SemaphoreType.DMA,
  ]
  if aux:
    scratch[11:11] = [
        pltpu.VMEM((RC, H), jnp.float32),
        pltpu.VMEM((RC, H), jnp.float32),
    ]

  def body(*refs):
    if aux:
      (cols_h, vals_h, rows_h, u_h, p_h, oin_h, coef_h, meta_h, xa_h, ta_h,
       tnew_h, onew_h,
       acc, cbuf, vbuf, rbuf, gbuf, zbuf, svm, uvm, pvm, ovm, twm,
       xvm, tvm, coefv, metav, sem) = refs
    else:
      (cols_h, vals_h, rows_h, u_h, p_h, oin_h, coef_h, meta_h,
       tnew_h, onew_h,
       acc, cbuf, vbuf, rbuf, gbuf, zbuf, svm, uvm, pvm, ovm, twm,
       coefv, metav, sem) = refs

    c = lax.axis_index("c")
    s = lax.axis_index("s")

    pltpu.sync_copy(coef_h, coefv)
    pltpu.sync_copy(meta_h, metav)
    cvec = coefv[...]
    a_, b_, d_ = cvec[0], cvec[1], cvec[2]
    f0, f1, f2 = cvec[3], cvec[4], cvec[5]
    f3, f4 = cvec[6], cvec[7]
    mvec = metav[...]
    split = mvec[0]
    etot = mvec[1]

    base_c = jnp.where(c == 0, 0, split)
    cnt = jnp.where(c == 0, split, etot - split)
    per = ((cnt + NTILE - 1) // NTILE + 7) // 8 * 8
    my0 = base_c + s * per
    mycnt = jnp.clip(cnt - s * per, 0, per)
    nch = (mycnt + G - 1) // G

    r0 = jnp.minimum(s * PT, HALF)
    r1 = jnp.minimum(r0 + PT, HALF)

    def zrow(i, _):
      for q in range(NQ):
        zbuf[i, pl.ds(q * 16, 16)] = jnp.zeros((16,), jnp.float32)
      return 0
    lax.fori_loop(0, RC, zrow, 0)

    nrz = (r1 - r0 + RC - 1) // RC

    def zc(i, _):
      ri = pl.multiple_of(jnp.minimum(r0 + i * RC, r1 - RC), 8)
      pltpu.sync_copy(zbuf, acc.at[pl.ds(ri, RC)])
      return 0
    lax.fori_loop(0, nrz, zc, 0)
    plsc.subcore_barrier()

    lanes0 = lax.iota(jnp.int32, 16)

    def chunk(i, _):
      off = pl.multiple_of(my0 + i * G, 8)
      pltpu.sync_copy(cols_h.at[pl.ds(off, G)], cbuf)
      pltpu.sync_copy(vals_h.at[pl.ds(off, G)], vbuf)
      pltpu.sync_copy(rows_h.at[pl.ds(off, G)], rbuf)
      pltpu.async_copy(u_h.at[cbuf], gbuf, sem).wait()
      rem = mycnt - i * G

      def grp(j, _):
        vv = vbuf[pl.ds(j * 16, 16)]
        vv = jnp.where(j * 16 + lanes0 < rem, vv, 0.0)
        for l in range(16):
          val = vv[l]
          g = j * 16 + l
          for q in range(NQ):
            sl = pl.ds(q * 16, 16)
            gbuf[g, sl] = gbuf[g, sl] * val
        return 0
      lax.fori_loop(0, G // 16, grp, 0)
      pltpu.sync_copy(gbuf, acc.at[rbuf], add=True)
      return 0
    lax.fori_loop(0, nch, chunk, 0)
    plsc.subcore_barrier()

    gbase = c * HALF
    nrc = (r1 - r0 + RC - 1) // RC

    def cc(i, _):
      ri = pl.multiple_of(jnp.minimum(r0 + i * RC, r1 - RC), 8)
      grow = pl.multiple_of(gbase + ri, 8)
      pltpu.sync_copy(acc.at[pl.ds(ri, RC)], svm)
      pltpu.sync_copy(u_h.at[pl.ds(grow, RC)], uvm)
      pltpu.sync_copy(p_h.at[pl.ds(grow, RC)], pvm)
      pltpu.sync_copy(oin_h.at[pl.ds(grow, RC)], ovm)
      if aux:
        pltpu.sync_copy(xa_h.at[pl.ds(grow, RC)], xvm)
        pltpu.sync_copy(ta_h.at[pl.ds(grow, RC)], tvm)

      def crow(r, _):
        for q in range(NQ):
          sl = pl.ds(q * 16, 16)
          uv = uvm[r, sl]
          t = a_ * svm[r, sl] + b_ * uv + d_ * pvm[r, sl]
          o = f0 * ovm[r, sl] + f1 * uv + f2 * t
          if aux:
            o = o + f3 * xvm[r, sl] + f4 * tvm[r, sl]
          twm[r, sl] = t
          ovm[r, sl] = o
        return 0
      lax.fori_loop(0, RC, crow, 0)
      pltpu.sync_copy(twm, tnew_h.at[pl.ds(grow, RC)])
      pltpu.sync_copy(ovm, onew_h.at[pl.ds(grow, RC)])
      return 0
    lax.fori_loop(0, nrc, cc, 0)

  out_type = [jax.ShapeDtypeStruct((N, H), jnp.float32)] * 2
  return pl.kernel(body, out_type=out_type, mesh=mesh,
                   scratch_types=scratch,
                   name="cheb_spmm_aux" if aux else "cheb_spmm")


def kernel(x, edge_index, edge_vals, poly_logits, hp_alpha):
  N, H = x.shape
  E = edge_index.shape[1]
  K = poly_logits.shape[0] - 1
  HALF = N // 2

  rows = edge_index[0]
  cols = edge_index[1]
  order = jnp.argsort(rows)
  rows_s = jnp.take(rows, order)
  cols_s = jnp.take(cols, order)
  vals_s = jnp.take(edge_vals, order)

  split = jnp.searchsorted(rows_s, HALF).astype(jnp.int32)
  pad = (8 - split % 8) % 8
  etot = E + 8
  ep = etot + G
  idx = jnp.arange(E, dtype=jnp.int32)
  pos = idx + jnp.where(idx >= split, pad, 0)
  colsp = jnp.zeros((ep,), jnp.int32).at[pos].set(cols_s)
  valsp = jnp.zeros((ep,), jnp.float32).at[pos].set(vals_s)
  rloc = jnp.where(rows_s >= HALF, rows_s - HALF, rows_s)
  rowsp = jnp.zeros((ep,), jnp.int32).at[pos].set(rloc)
  meta = (jnp.zeros((16,), jnp.int32)
          .at[0].set(split + pad).at[1].set(etot))

  rp = jnp.searchsorted(rows_s, jnp.arange(N + 1, dtype=jnp.int32))
  rp = rp.astype(jnp.float32)
  l2 = 80 * 128
  rp_pad = jnp.full((l2 + 1,), rp[-1], jnp.float32).at[:N + 1].set(rp)
  dm = _degmax_tc(rp_pad[:l2].reshape(80, 128),
                  rp_pad[1:l2 + 1].reshape(80, 128))
  lam = 2.0 * dm[0, 0]
  c2 = 2.0 / lam

  w = jax.nn.softmax(poly_logits)
  alpha = hp_alpha.astype(jnp.float32)

  def mk(a, b, d, f0, f1, f2, f3=0.0, f4=0.0):
    vals = jnp.stack([jnp.asarray(v, jnp.float32) * jnp.ones((), jnp.float32)
                      for v in (a, b, d, f0, f1, f2, f3, f4)])
    return jnp.concatenate([vals, jnp.zeros((8,), jnp.float32)])

  spmm = _make_spmm(N, H, ep, False)
  spmm_aux = _make_spmm(N, H, ep, True)

  t1, out = spmm(colsp, valsp, rowsp, x, x, x,
                 mk(c2, -1.0, 0.0, 0.0, w[0], w[1]), meta)
  tprev, tcur = x, t1
  for k in range(1, K):
    if k < K - 1:
      coef = mk(2.0 * c2, -2.0, -1.0, 1.0, 0.0, w[k + 1])
      tnext, out = spmm(colsp, valsp, rowsp, tcur, tprev, out, coef, meta)
    else:
      coef = mk(2.0 * c2, -2.0, -1.0, -1.0, 0.0, -w[k + 1],
                alpha * 0.5, -alpha * 0.5)
      tnext, out = spmm_aux(colsp, valsp, rowsp, tcur, tprev, out, coef,
                            meta, x, t1)
    tprev, tcur = tcur, tnext
  return out


# speedup vs baseline: 4.7188x; 1.9691x over previous
import functools

import jax
import jax.numpy as jnp
from jax import lax
from jax.experimental import pallas as pl
from jax.experimental.pallas import tpu as pltpu
from jax.experimental.pallas import tpu_sc as plsc

G = 128
RC = 32
NTILE = 16
BR = 1024


def _edge_split(e2, etot, c, s):
  base_c = jnp.where(c == 0, 0, e2)
  cnt = jnp.where(c == 0, e2, etot - e2)
  per = ((cnt + NTILE - 1) // NTILE + 7) // 8 * 8
  my0 = base_c + s * per
  mycnt = jnp.clip(cnt - s * per, 0, per)
  return my0, mycnt


@functools.lru_cache(maxsize=None)
def _make_spmm(npad, H, ep, e2, etot):
  PT = npad // NTILE
  NQ = H // 16
  mesh = plsc.VectorSubcoreMesh(core_axis_name="c", subcore_axis_name="s")

  scratch = [
      pltpu.VMEM_SHARED((npad, H), jnp.float32),
      pltpu.VMEM((G,), jnp.int32),
      pltpu.VMEM((G,), jnp.float32),
      pltpu.VMEM((G,), jnp.int32),
      pltpu.VMEM((G, H), jnp.float32),
      pltpu.VMEM((RC, H), jnp.float32),
      pltpu.SemaphoreType.DMA,
  ]

  def body(cols_h, vals_h, rows_h, u_h, p0_h, p1_h,
           acc, cbuf, vbuf, rbuf, gbuf, zbuf, sem):
    c = lax.axis_index("c")
    s = lax.axis_index("s")
    my0, mycnt = _edge_split(e2, etot, c, s)
    nch = (mycnt + G - 1) // G

    def zrow(i, _):
      for q in range(NQ):
        zbuf[i, pl.ds(q * 16, 16)] = jnp.zeros((16,), jnp.float32)
      return 0
    lax.fori_loop(0, RC, zrow, 0)

    r0 = s * PT

    def zc(i, _):
      ri = pl.multiple_of(r0 + i * RC, 8)
      pltpu.sync_copy(zbuf, acc.at[pl.ds(ri, RC)])
      return 0
    lax.fori_loop(0, PT // RC, zc, 0)
    plsc.subcore_barrier()

    lanes0 = lax.iota(jnp.int32, 16)

    def chunk(i, _):
      off = pl.multiple_of(my0 + i * G, 8)
      pltpu.sync_copy(cols_h.at[pl.ds(off, G)], cbuf)
      pltpu.sync_copy(vals_h.at[pl.ds(off, G)], vbuf)
      pltpu.sync_copy(rows_h.at[pl.ds(off, G)], rbuf)
      pltpu.async_copy(u_h.at[cbuf], gbuf, sem).wait()
      rem = mycnt - i * G

      def grp(j, _):
        vv = vbuf[pl.ds(j * 16, 16)]
        vv = jnp.where(j * 16 + lanes0 < rem, vv, 0.0)
        for l in range(16):
          val = vv[l]
          g = j * 16 + l
          for q in range(NQ):
            sl = pl.ds(q * 16, 16)
            gbuf[g, sl] = gbuf[g, sl] * val
        return 0
      lax.fori_loop(0, G // 16, grp, 0)
      pltpu.sync_copy(gbuf, acc.at[rbuf], add=True)
      return 0
    lax.fori_loop(0, nch, chunk, 0)
    plsc.subcore_barrier()

    @pl.when(jnp.logical_and(s == 0, c == 0))
    def _():
      pltpu.sync_copy(acc, p0_h)

    @pl.when(jnp.logical_and(s == 0, c == 1))
    def _():
      pltpu.sync_copy(acc, p1_h)

  out_type = [jax.ShapeDtypeStruct((npad, H), jnp.float32)] * 2
  return pl.kernel(body, out_type=out_type, mesh=mesh,
                   scratch_types=scratch, name="cheb_spmm")


@functools.lru_cache(maxsize=None)
def _make_deg(npad, ep, e2, etot):
  mesh = plsc.VectorSubcoreMesh(core_axis_name="c", subcore_axis_name="s")
  PT = npad // NTILE

  scratch = [
      pltpu.VMEM_SHARED((npad,), jnp.float32),
      pltpu.VMEM((G,), jnp.int32),
      pltpu.VMEM((G,), jnp.float32),
      pltpu.VMEM((RC * 16,), jnp.float32),
  ]

  def body(rows_h, d0_h, d1_h, accd, rbuf, obuf, zbuf):
    c = lax.axis_index("c")
    s = lax.axis_index("s")
    my0, mycnt = _edge_split(e2, etot, c, s)
    nch = (mycnt + G - 1) // G

    def zrow(i, _):
      zbuf[pl.ds(i * 16, 16)] = jnp.zeros((16,), jnp.float32)
      return 0
    lax.fori_loop(0, RC, zrow, 0)
    r0 = s * PT

    def zc(i, _):
      ri = pl.multiple_of(r0 + i * RC * 16, 8)
      pltpu.sync_copy(zbuf, accd.at[pl.ds(ri, RC * 16)])
      return 0
    lax.fori_loop(0, PT // (RC * 16), zc, 0)
    plsc.subcore_barrier()

    lanes0 = lax.iota(jnp.int32, 16)

    def chunk(i, _):
      off = pl.multiple_of(my0 + i * G, 8)
      pltpu.sync_copy(rows_h.at[pl.ds(off, G)], rbuf)
      rem = mycnt - i * G
      for j in range(G // 16):
        ones = jnp.where(j * 16 + lanes0 < rem, 1.0, 0.0)
        obuf[pl.ds(j * 16, 16)] = ones
      pltpu.sync_copy(obuf, accd.at[rbuf], add=True)
      return 0
    lax.fori_loop(0, nch, chunk, 0)
    plsc.subcore_barrier()

    @pl.when(jnp.logical_and(s == 0, c == 0))
    def _():
      pltpu.sync_copy(accd, d0_h)

    @pl.when(jnp.logical_and(s == 0, c == 1))
    def _():
      pltpu.sync_copy(accd, d1_h)

  out_type = [jax.ShapeDtypeStruct((npad,), jnp.float32)] * 2
  return pl.kernel(body, out_type=out_type, mesh=mesh,
                   scratch_types=scratch, name="deg_count")


def _degmax_tc(d0, d1):
  def body(a_ref, b_ref, o_ref):
    m = jnp.max(a_ref[...] + b_ref[...])
    o_ref[...] = jnp.full((8, 128), m, jnp.float32)
  return pl.pallas_call(
      body, out_shape=jax.ShapeDtypeStruct((8, 128), jnp.float32))(d0, d1)


@functools.lru_cache(maxsize=None)
def _make_merge(npad, H, aux):
  grid = npad // BR

  def body(*refs):
    if aux:
      coef, p0, p1, u, p, o, xa, ta, t_out, o_out = refs
    else:
      coef, p0, p1, u, p, o, t_out, o_out = refs
    a_, b_, d_ = coef[0], coef[1], coef[2]
    f0, f1, f2 = coef[3], coef[4], coef[5]
    uv = u[...]
    t = a_ * (p0[...] + p1[...]) + b_ * uv + d_ * p[...]
    oo = f0 * o[...] + f1 * uv + f2 * t
    if aux:
      oo = oo + coef[6] * xa[...] + coef[7] * ta[...]
    t_out[...] = t
    o_out[...] = oo

  narr = 7 if aux else 5
  bspec = pl.BlockSpec((BR, H), lambda i: (i, 0))
  return pl.pallas_call(
      body,
      grid=(grid,),
      in_specs=[pl.BlockSpec(memory_space=pltpu.SMEM)] + [bspec] * narr,
      out_specs=[bspec, bspec],
      out_shape=[jax.ShapeDtypeStruct((npad, H), jnp.float32)] * 2,
  )


def kernel(x, edge_index, edge_vals, poly_logits, hp_alpha):
  N, H = x.shape
  E = edge_index.shape[1]
  K = poly_logits.shape[0] - 1
  npad = -(-N // BR) * BR

  rows = edge_index[0]
  cols = edge_index[1]
  zi = jnp.zeros((G,), jnp.int32)
  rows_p = jnp.concatenate([rows, zi])
  cols_p = jnp.concatenate([cols, zi])
  vals_p = jnp.concatenate([edge_vals, jnp.zeros((G,), jnp.float32)])
  ep = E + G
  e2 = (-(-E // 2) + 7) // 8 * 8

  xp = jnp.concatenate([x, jnp.zeros((npad - N, H), jnp.float32)])

  deg = _make_deg(npad, ep, e2, E)
  d0, d1 = deg(rows_p)
  dm = _degmax_tc(d0.reshape(npad // 128, 128), d1.reshape(npad // 128, 128))
  lam = 2.0 * dm[0, 0]
  c2 = 2.0 / lam

  w = jax.nn.softmax(poly_logits)
  alpha = hp_alpha.astype(jnp.float32)

  def mk(a, b, d, f0, f1, f2, f3=0.0, f4=0.0):
    return jnp.stack([jnp.asarray(v, jnp.float32) * jnp.ones((), jnp.float32)
                      for v in (a, b, d, f0, f1, f2, f3, f4)])

  spmm = _make_spmm(npad, H, ep, e2, E)
  merge = _make_merge(npad, H, False)
  merge_aux = _make_merge(npad, H, True)

  p0, p1 = spmm(cols_p, vals_p, rows_p, xp)
  t1, out = merge(mk(c2, -1.0, 0.0, 0.0, w[0], w[1]), p0, p1, xp, xp, xp)
  tprev, tcur = xp, t1
  for k in range(1, K):
    p0, p1 = spmm(cols_p, vals_p, rows_p, tcur)
    if k < K - 1:
      coef = mk(2.0 * c2, -2.0, -1.0, 1.0, 0.0, w[k + 1])
      tnext, out = merge(coef, p0, p1, tcur, tprev, out)
    else:
      coef = mk(2.0 * c2, -2.0, -1.0, -1.0, 0.0, -w[k + 1],
                alpha * 0.5, -alpha * 0.5)
      tnext, out = merge_aux(coef, p0, p1, tcur, tprev, out, xp, t1)
    tprev, tcur = tcur, tnext
  return out[:N]


# speedup vs baseline: 6.8342x; 1.4483x over previous
import functools

import jax
import jax.numpy as jnp
from jax import lax
from jax.experimental import pallas as pl
from jax.experimental.pallas import tpu as pltpu
from jax.experimental.pallas import tpu_sc as plsc

G = 128
SUPER = 2
RC = 32
NTILE = 16
BR = 1024


def _edge_split(e2, etot, c, s):
  base_c = jnp.where(c == 0, 0, e2)
  cnt = jnp.where(c == 0, e2, etot - e2)
  per = ((cnt + NTILE - 1) // NTILE + 7) // 8 * 8
  my0 = base_c + s * per
  mycnt = jnp.clip(cnt - s * per, 0, per)
  return my0, mycnt


@functools.lru_cache(maxsize=None)
def _make_spmm(npad, H, e2, etot, per0, per1):
  PT = npad // NTILE
  NQ = H // 16
  CH = SUPER * G
  mesh = plsc.VectorSubcoreMesh(core_axis_name="c", subcore_axis_name="s")

  scratch = [
      pltpu.VMEM_SHARED((npad, H), jnp.float32),
      pltpu.VMEM((SUPER * 3 * G,), jnp.int32),
      pltpu.VMEM((SUPER, G), jnp.int32),
      pltpu.VMEM((SUPER, G), jnp.int32),
      pltpu.VMEM((SUPER, G, H), jnp.float32),
      pltpu.VMEM((RC, H), jnp.float32),
      pltpu.SemaphoreType.DMA,
  ]

  def body(packed_h, u_h, p0_h, p1_h,
           acc, pbuf, cbuf, rbuf, gbuf, zbuf, sem):
    c = lax.axis_index("c")
    s = lax.axis_index("s")
    per = jnp.where(c == 0, per0, per1)
    my0 = jnp.where(c == 0, 0, e2) + s * per
    cnt = jnp.where(c == 0, e2, etot - e2)
    mycnt = jnp.clip(cnt - s * per, 0, per)
    nch = (mycnt + CH - 1) // CH

    def zrow(i, _):
      for q in range(NQ):
        zbuf[i, pl.ds(q * 16, 16)] = jnp.zeros((16,), jnp.float32)
      return 0
    lax.fori_loop(0, RC, zrow, 0)

    r0 = s * PT

    def zc(i, _):
      ri = pl.multiple_of(r0 + i * RC, 8)
      pltpu.sync_copy(zbuf, acc.at[pl.ds(ri, RC)])
      return 0
    lax.fori_loop(0, PT // RC, zc, 0)
    plsc.subcore_barrier()

    lanes0 = lax.iota(jnp.int32, 16)

    def chunk(i, _):
      off3 = pl.multiple_of((my0 // G + i * SUPER) * 3 * G, 8)
      pltpu.sync_copy(packed_h.at[pl.ds(off3, SUPER * 3 * G)], pbuf)
      for q in range(SUPER):
        for j in range(8):
          sl = pl.ds(j * 16, 16)
          cbuf[q, sl] = pbuf[pl.ds((q * 3 + 0) * G + j * 16, 16)]
      descs = [
          pltpu.async_copy(u_h.at[cbuf.at[q]], gbuf.at[q], sem)
          for q in range(SUPER)
      ]
      for q in range(SUPER):
        for j in range(8):
          sl = pl.ds(j * 16, 16)
          rbuf[q, sl] = pbuf[pl.ds((q * 3 + 2) * G + j * 16, 16)]
      for d in descs:
        d.wait()
      base = mycnt - i * CH
      for q in range(SUPER):
        remq = base - q * G

        def grp(j, _, q=q, remq=remq):
          vi = pbuf[pl.ds((q * 3 + 1) * G + j * 16, 16)]
          vv = lax.bitcast_convert_type(vi, jnp.float32)
          vv = jnp.where(j * 16 + lanes0 < remq, vv, 0.0)
          for l in range(16):
            val = vv[l]
            g = j * 16 + l
            for qq in range(NQ):
              sl = pl.ds(qq * 16, 16)
              gbuf[q, g, sl] = gbuf[q, g, sl] * val
          return 0
        lax.fori_loop(0, G // 16, grp, 0)
      for q in range(SUPER):
        pltpu.sync_copy(gbuf.at[q], acc.at[rbuf.at[q]], add=True)
      return 0
    lax.fori_loop(0, nch, chunk, 0)
    plsc.subcore_barrier()

    @pl.when(jnp.logical_and(s == 0, c == 0))
    def _():
      pltpu.sync_copy(acc, p0_h)

    @pl.when(jnp.logical_and(s == 0, c == 1))
    def _():
      pltpu.sync_copy(acc, p1_h)

  out_type = [jax.ShapeDtypeStruct((npad, H), jnp.float32)] * 2
  return pl.kernel(body, out_type=out_type, mesh=mesh,
                   scratch_types=scratch, name="cheb_spmm")


def _edge_pad_len(E, e2):
  per0 = -(-(-(-e2 // NTILE)) // G) * G
  per1 = -(-(-(-(E - e2) // NTILE)) // G) * G
  end0 = NTILE * per0 + SUPER * G
  end1 = e2 + NTILE * per1 + SUPER * G
  n = max(end0, end1, E)
  return -(-n // G) * G, per0, per1


@functools.lru_cache(maxsize=None)
def _make_deg(npad, ep, e2, etot):
  mesh = plsc.VectorSubcoreMesh(core_axis_name="c", subcore_axis_name="s")
  PT = npad // NTILE

  scratch = [
      pltpu.VMEM_SHARED((npad,), jnp.float32),
      pltpu.VMEM((G,), jnp.int32),
      pltpu.VMEM((G,), jnp.float32),
      pltpu.VMEM((RC * 16,), jnp.float32),
  ]

  def body(rows_h, d0_h, d1_h, accd, rbuf, obuf, zbuf):
    c = lax.axis_index("c")
    s = lax.axis_index("s")
    my0, mycnt = _edge_split(e2, etot, c, s)
    nch = (mycnt + G - 1) // G

    def zrow(i, _):
      zbuf[pl.ds(i * 16, 16)] = jnp.zeros((16,), jnp.float32)
      return 0
    lax.fori_loop(0, RC, zrow, 0)
    r0 = s * PT

    def zc(i, _):
      ri = pl.multiple_of(r0 + i * RC * 16, 8)
      pltpu.sync_copy(zbuf, accd.at[pl.ds(ri, RC * 16)])
      return 0
    lax.fori_loop(0, PT // (RC * 16), zc, 0)
    plsc.subcore_barrier()

    lanes0 = lax.iota(jnp.int32, 16)

    def chunk(i, _):
      off = pl.multiple_of(my0 + i * G, 8)
      pltpu.sync_copy(rows_h.at[pl.ds(off, G)], rbuf)
      rem = mycnt - i * G
      for j in range(G // 16):
        ones = jnp.where(j * 16 + lanes0 < rem, 1.0, 0.0)
        obuf[pl.ds(j * 16, 16)] = ones
      pltpu.sync_copy(obuf, accd.at[rbuf], add=True)
      return 0
    lax.fori_loop(0, nch, chunk, 0)
    plsc.subcore_barrier()

    @pl.when(jnp.logical_and(s == 0, c == 0))
    def _():
      pltpu.sync_copy(accd, d0_h)

    @pl.when(jnp.logical_and(s == 0, c == 1))
    def _():
      pltpu.sync_copy(accd, d1_h)

  out_type = [jax.ShapeDtypeStruct((npad,), jnp.float32)] * 2
  return pl.kernel(body, out_type=out_type, mesh=mesh,
                   scratch_types=scratch, name="deg_count")


def _degmax_tc(d0, d1):
  def body(a_ref, b_ref, o_ref):
    m = jnp.max(a_ref[...] + b_ref[...])
    o_ref[...] = jnp.full((8, 128), m, jnp.float32)
  return pl.pallas_call(
      body, out_shape=jax.ShapeDtypeStruct((8, 128), jnp.float32))(d0, d1)


@functools.lru_cache(maxsize=None)
def _make_merge(npad, H, aux):
  grid = npad // BR

  def body(*refs):
    if aux:
      coef, p0, p1, u, p, o, xa, ta, t_out, o_out = refs
    else:
      coef, p0, p1, u, p, o, t_out, o_out = refs
    a_, b_, d_ = coef[0], coef[1], coef[2]
    f0, f1, f2 = coef[3], coef[4], coef[5]
    uv = u[...]
    t = a_ * (p0[...] + p1[...]) + b_ * uv + d_ * p[...]
    oo = f0 * o[...] + f1 * uv + f2 * t
    if aux:
      oo = oo + coef[6] * xa[...] + coef[7] * ta[...]
    t_out[...] = t
    o_out[...] = oo

  narr = 7 if aux else 5
  bspec = pl.BlockSpec((BR, H), lambda i: (i, 0))
  return pl.pallas_call(
      body,
      grid=(grid,),
      in_specs=[pl.BlockSpec(memory_space=pltpu.SMEM)] + [bspec] * narr,
      out_specs=[bspec, bspec],
      out_shape=[jax.ShapeDtypeStruct((npad, H), jnp.float32)] * 2,
  )


def kernel(x, edge_index, edge_vals, poly_logits, hp_alpha):
  N, H = x.shape
  E = edge_index.shape[1]
  K = poly_logits.shape[0] - 1
  npad = -(-N // BR) * BR

  rows = edge_index[0]
  cols = edge_index[1]
  e2 = -(-(-(-E // 2)) // G) * G
  nbg, per0, per1 = _edge_pad_len(E, e2)
  zpad = jnp.zeros((nbg - E,), jnp.int32)
  rows_p = jnp.concatenate([rows, zpad])
  cols_p = jnp.concatenate([cols, zpad])
  vbits = lax.bitcast_convert_type(edge_vals, jnp.int32)
  vals_p = jnp.concatenate([vbits, zpad])
  nb = nbg // G
  packed = jnp.stack([cols_p.reshape(nb, G), vals_p.reshape(nb, G),
                      rows_p.reshape(nb, G)], axis=1).reshape(-1)
  ep = nbg

  xp = jnp.concatenate([x, jnp.zeros((npad - N, H), jnp.float32)])

  deg = _make_deg(npad, ep, e2, E)
  d0, d1 = deg(rows_p)
  dm = _degmax_tc(d0.reshape(npad // 128, 128), d1.reshape(npad // 128, 128))
  lam = 2.0 * dm[0, 0]
  c2 = 2.0 / lam

  w = jax.nn.softmax(poly_logits)
  alpha = hp_alpha.astype(jnp.float32)

  def mk(a, b, d, f0, f1, f2, f3=0.0, f4=0.0):
    return jnp.stack([jnp.asarray(v, jnp.float32) * jnp.ones((), jnp.float32)
                      for v in (a, b, d, f0, f1, f2, f3, f4)])

  spmm = _make_spmm(npad, H, e2, E, per0, per1)
  merge = _make_merge(npad, H, False)
  merge_aux = _make_merge(npad, H, True)

  p0, p1 = spmm(packed, xp)
  t1, out = merge(mk(c2, -1.0, 0.0, 0.0, w[0], w[1]), p0, p1, xp, xp, xp)
  tprev, tcur = xp, t1
  for k in range(1, K):
    p0, p1 = spmm(packed, tcur)
    if k < K - 1:
      coef = mk(2.0 * c2, -2.0, -1.0, 1.0, 0.0, w[k + 1])
      tnext, out = merge(coef, p0, p1, tcur, tprev, out)
    else:
      coef = mk(2.0 * c2, -2.0, -1.0, -1.0, 0.0, -w[k + 1],
                alpha * 0.5, -alpha * 0.5)
      tnext, out = merge_aux(coef, p0, p1, tcur, tprev, out, xp, t1)
    tprev, tcur = tcur, tnext
  return out[:N]


# speedup vs baseline: 9.1973x; 1.3458x over previous
import functools

import jax
import jax.numpy as jnp
from jax import lax
from jax.experimental import pallas as pl
from jax.experimental.pallas import tpu as pltpu
from jax.experimental.pallas import tpu_sc as plsc

G = 128
SUPER = 2
RC = 32
NTILE = 16
BR = 1024


def _edge_split(e2, etot, c, s):
  base_c = jnp.where(c == 0, 0, e2)
  cnt = jnp.where(c == 0, e2, etot - e2)
  per = ((cnt + NTILE - 1) // NTILE + 7) // 8 * 8
  my0 = base_c + s * per
  mycnt = jnp.clip(cnt - s * per, 0, per)
  return my0, mycnt


@functools.lru_cache(maxsize=None)
def _make_spmm(npad, H, e2, etot, per0, per1):
  PT = npad // NTILE
  NQ = H // 16
  CH = SUPER * G
  mesh = plsc.VectorSubcoreMesh(core_axis_name="c", subcore_axis_name="s")

  scratch = [
      pltpu.VMEM_SHARED((npad, H), jnp.float32),
      pltpu.VMEM((2 * 3 * G,), jnp.int32),
      pltpu.VMEM((2, G), jnp.int32),
      pltpu.VMEM((2, G, H), jnp.float32),
      pltpu.VMEM((RC, H), jnp.float32),
      pltpu.SemaphoreType.DMA,
      pltpu.SemaphoreType.DMA,
      pltpu.SemaphoreType.DMA,
      pltpu.SemaphoreType.DMA,
      pltpu.SemaphoreType.DMA,
  ]

  def body(packed_h, u_h, p0_h, p1_h,
           acc, pbuf, rbuf, gbuf, zbuf, semz, semi0, semi1, semg0, semg1):
    c = lax.axis_index("c")
    s = lax.axis_index("s")
    semi = [semi0, semi1]
    semg = [semg0, semg1]
    per = jnp.where(c == 0, per0, per1)
    my0 = jnp.where(c == 0, 0, e2) + s * per
    cnt = jnp.where(c == 0, e2, etot - e2)
    mycnt = jnp.clip(cnt - s * per, 0, per)
    nch = (mycnt + G - 1) // G

    def zrow(i, _):
      for q in range(NQ):
        zbuf[i, pl.ds(q * 16, 16)] = jnp.zeros((16,), jnp.float32)
      return 0
    lax.fori_loop(0, RC, zrow, 0)

    r0 = s * PT

    def zfire(i, _):
      ri = pl.multiple_of(r0 + i * RC, 8)
      pltpu.async_copy(zbuf, acc.at[pl.ds(ri, RC)], semz)
      return 0
    lax.fori_loop(0, PT // RC, zfire, 0)

    def zdrain(i, _):
      ri = pl.multiple_of(r0 + i * RC, 8)
      pltpu.make_async_copy(zbuf, acc.at[pl.ds(ri, RC)], semz).wait()
      return 0
    lax.fori_loop(0, PT // RC, zdrain, 0)
    plsc.subcore_barrier()

    lanes0 = lax.iota(jnp.int32, 16)

    def pslice(b, part, n=3 * G):
      return pbuf.at[pl.ds(b * 3 * G + part * G, n)]

    def idx_src(i):
      off3 = pl.multiple_of((my0 // G + i) * 3 * G, 8)
      return packed_h.at[pl.ds(off3, 3 * G)]

    def idx_issue(i, b):
      pltpu.async_copy(idx_src(i), pslice(b, 0), semi[b])

    def idx_wait(i, b):
      pltpu.make_async_copy(idx_src(i), pslice(b, 0), semi[b]).wait()

    def gather_issue(b):
      pltpu.async_copy(u_h.at[pslice(b, 0, G)], gbuf.at[b], semg[b])

    def gather_wait(b):
      pltpu.make_async_copy(u_h.at[pslice(b, 0, G)], gbuf.at[b],
                            semg[b]).wait()

    def process(i, b):
      rem = mycnt - i * G

      def grp(j, _):
        vi = pbuf[pl.ds(b * 3 * G + G + j * 16, 16)]
        vv = lax.bitcast_convert_type(vi, jnp.float32)
        vv = jnp.where(j * 16 + lanes0 < rem, vv, 0.0)
        for l in range(16):
          val = vv[l]
          g = j * 16 + l
          for qq in range(NQ):
            sl = pl.ds(qq * 16, 16)
            gbuf[b, g, sl] = gbuf[b, g, sl] * val
        return 0
      lax.fori_loop(0, G // 16, grp, 0)
      for j in range(8):
        sl = pl.ds(j * 16, 16)
        rbuf[b, sl] = pbuf[pl.ds(b * 3 * G + 2 * G + j * 16, 16)]
      pltpu.sync_copy(gbuf.at[b], acc.at[rbuf.at[b]], add=True)

    @pl.when(nch > 0)
    def _():
      pltpu.sync_copy(idx_src(0), pslice(0, 0))
      gather_issue(0)

    @pl.when(nch > 1)
    def _():
      idx_issue(1, 1)

    def pair(p, _):
      i0, i1, i2, i3 = 2 * p, 2 * p + 1, 2 * p + 2, 2 * p + 3

      @pl.when(i1 < nch)
      def _():
        idx_wait(i1, 1)
        gather_issue(1)

      gather_wait(0)
      process(i0, 0)

      @pl.when(i2 < nch)
      def _():
        idx_issue(i2, 0)
        idx_wait(i2, 0)
        gather_issue(0)

      @pl.when(i1 < nch)
      def _():
        gather_wait(1)
        process(i1, 1)

      @pl.when(i3 < nch)
      def _():
        idx_issue(i3, 1)
      return 0
    lax.fori_loop(0, (nch + 1) // 2, pair, 0)
    plsc.subcore_barrier()

    @pl.when(jnp.logical_and(s == 0, c == 0))
    def _():
      pltpu.sync_copy(acc, p0_h)

    @pl.when(jnp.logical_and(s == 0, c == 1))
    def _():
      pltpu.sync_copy(acc, p1_h)

  out_type = [jax.ShapeDtypeStruct((npad, H), jnp.float32)] * 2
  return pl.kernel(body, out_type=out_type, mesh=mesh,
                   scratch_types=scratch, name="cheb_spmm")


def _edge_pad_len(E, e2):
  per0 = -(-(-(-e2 // NTILE)) // G) * G
  per1 = -(-(-(-(E - e2) // NTILE)) // G) * G
  end0 = NTILE * per0 + SUPER * G
  end1 = e2 + NTILE * per1 + SUPER * G
  n = max(end0, end1, E)
  return -(-n // G) * G, per0, per1


@functools.lru_cache(maxsize=None)
def _make_deg(npad, ep, e2, etot):
  mesh = plsc.VectorSubcoreMesh(core_axis_name="c", subcore_axis_name="s")
  PT = npad // NTILE

  scratch = [
      pltpu.VMEM_SHARED((npad,), jnp.float32),
      pltpu.VMEM((G,), jnp.int32),
      pltpu.VMEM((G,), jnp.float32),
      pltpu.VMEM((RC * 16,), jnp.float32),
  ]

  def body(rows_h, d0_h, d1_h, accd, rbuf, obuf, zbuf):
    c = lax.axis_index("c")
    s = lax.axis_index("s")
    my0, mycnt = _edge_split(e2, etot, c, s)
    nch = (mycnt + G - 1) // G

    def zrow(i, _):
      zbuf[pl.ds(i * 16, 16)] = jnp.zeros((16,), jnp.float32)
      return 0
    lax.fori_loop(0, RC, zrow, 0)
    r0 = s * PT

    def zc(i, _):
      ri = pl.multiple_of(r0 + i * RC * 16, 8)
      pltpu.sync_copy(zbuf, accd.at[pl.ds(ri, RC * 16)])
      return 0
    lax.fori_loop(0, PT // (RC * 16), zc, 0)
    plsc.subcore_barrier()

    lanes0 = lax.iota(jnp.int32, 16)

    def chunk(i, _):
      off = pl.multiple_of(my0 + i * G, 8)
      pltpu.sync_copy(rows_h.at[pl.ds(off, G)], rbuf)
      rem = mycnt - i * G
      for j in range(G // 16):
        ones = jnp.where(j * 16 + lanes0 < rem, 1.0, 0.0)
        obuf[pl.ds(j * 16, 16)] = ones
      pltpu.sync_copy(obuf, accd.at[rbuf], add=True)
      return 0
    lax.fori_loop(0, nch, chunk, 0)
    plsc.subcore_barrier()

    @pl.when(jnp.logical_and(s == 0, c == 0))
    def _():
      pltpu.sync_copy(accd, d0_h)

    @pl.when(jnp.logical_and(s == 0, c == 1))
    def _():
      pltpu.sync_copy(accd, d1_h)

  out_type = [jax.ShapeDtypeStruct((npad,), jnp.float32)] * 2
  return pl.kernel(body, out_type=out_type, mesh=mesh,
                   scratch_types=scratch, name="deg_count")


def _degmax_tc(d0, d1):
  def body(a_ref, b_ref, o_ref):
    m = jnp.max(a_ref[...] + b_ref[...])
    o_ref[...] = jnp.full((8, 128), m, jnp.float32)
  return pl.pallas_call(
      body, out_shape=jax.ShapeDtypeStruct((8, 128), jnp.float32))(d0, d1)


@functools.lru_cache(maxsize=None)
def _make_merge(npad, H, aux):
  grid = npad // BR

  def body(*refs):
    if aux:
      coef, p0, p1, u, p, o, xa, ta, t_out, o_out = refs
    else:
      coef, p0, p1, u, p, o, t_out, o_out = refs
    a_, b_, d_ = coef[0], coef[1], coef[2]
    f0, f1, f2 = coef[3], coef[4], coef[5]
    uv = u[...]
    t = a_ * (p0[...] + p1[...]) + b_ * uv + d_ * p[...]
    oo = f0 * o[...] + f1 * uv + f2 * t
    if aux:
      oo = oo + coef[6] * xa[...] + coef[7] * ta[...]
    t_out[...] = t
    o_out[...] = oo

  narr = 7 if aux else 5
  bspec = pl.BlockSpec((BR, H), lambda i: (i, 0))
  return pl.pallas_call(
      body,
      grid=(grid,),
      in_specs=[pl.BlockSpec(memory_space=pltpu.SMEM)] + [bspec] * narr,
      out_specs=[bspec, bspec],
      out_shape=[jax.ShapeDtypeStruct((npad, H), jnp.float32)] * 2,
  )


def kernel(x, edge_index, edge_vals, poly_logits, hp_alpha):
  N, H = x.shape
  E = edge_index.shape[1]
  K = poly_logits.shape[0] - 1
  npad = -(-N // BR) * BR

  rows = edge_index[0]
  cols = edge_index[1]
  e2 = -(-(-(-E // 2)) // G) * G
  nbg, per0, per1 = _edge_pad_len(E, e2)
  zpad = jnp.zeros((nbg - E,), jnp.int32)
  rows_p = jnp.concatenate([rows, zpad])
  cols_p = jnp.concatenate([cols, zpad])
  vbits = lax.bitcast_convert_type(edge_vals, jnp.int32)
  vals_p = jnp.concatenate([vbits, zpad])
  nb = nbg // G
  packed = jnp.stack([cols_p.reshape(nb, G), vals_p.reshape(nb, G),
                      rows_p.reshape(nb, G)], axis=1).reshape(-1)
  ep = nbg

  xp = jnp.concatenate([x, jnp.zeros((npad - N, H), jnp.float32)])

  deg = _make_deg(npad, ep, e2, E)
  d0, d1 = deg(rows_p)
  dm = _degmax_tc(d0.reshape(npad // 128, 128), d1.reshape(npad // 128, 128))
  lam = 2.0 * dm[0, 0]
  c2 = 2.0 / lam

  w = jax.nn.softmax(poly_logits)
  alpha = hp_alpha.astype(jnp.float32)

  def mk(a, b, d, f0, f1, f2, f3=0.0, f4=0.0):
    return jnp.stack([jnp.asarray(v, jnp.float32) * jnp.ones((), jnp.float32)
                      for v in (a, b, d, f0, f1, f2, f3, f4)])

  spmm = _make_spmm(npad, H, e2, E, per0, per1)
  merge = _make_merge(npad, H, False)
  merge_aux = _make_merge(npad, H, True)

  p0, p1 = spmm(packed, xp)
  t1, out = merge(mk(c2, -1.0, 0.0, 0.0, w[0], w[1]), p0, p1, xp, xp, xp)
  tprev, tcur = xp, t1
  for k in range(1, K):
    p0, p1 = spmm(packed, tcur)
    if k < K - 1:
      coef = mk(2.0 * c2, -2.0, -1.0, 1.0, 0.0, w[k + 1])
      tnext, out = merge(coef, p0, p1, tcur, tprev, out)
    else:
      coef = mk(2.0 * c2, -2.0, -1.0, -1.0, 0.0, -w[k + 1],
                alpha * 0.5, -alpha * 0.5)
      tnext, out = merge_aux(coef, p0, p1, tcur, tprev, out, xp, t1)
    tprev, tcur = tcur, tnext
  return out[:N]


# speedup vs baseline: 11.0830x; 1.2050x over previous
import functools

import jax
import jax.numpy as jnp
from jax import lax
from jax.experimental import pallas as pl
from jax.experimental.pallas import tpu as pltpu
from jax.experimental.pallas import tpu_sc as plsc

G = 128
SUPER = 2
RC = 32
NTILE = 16
BR = 1024


def _edge_split(e2, etot, c, s):
  base_c = jnp.where(c == 0, 0, e2)
  cnt = jnp.where(c == 0, e2, etot - e2)
  per = ((cnt + NTILE - 1) // NTILE + 7) // 8 * 8
  my0 = base_c + s * per
  mycnt = jnp.clip(cnt - s * per, 0, per)
  return my0, mycnt


@functools.lru_cache(maxsize=None)
def _make_spmm(npad, H, e2, etot, per0, per1):
  PT = npad // NTILE
  NQ = H // 16
  CH = SUPER * G
  mesh = plsc.VectorSubcoreMesh(core_axis_name="c", subcore_axis_name="s")

  scratch = [
      pltpu.VMEM_SHARED((npad, H), jnp.float32),
      pltpu.VMEM((2 * 3 * G,), jnp.int32),
      pltpu.VMEM((2, G), jnp.int32),
      pltpu.VMEM((2, G, H), jnp.float32),
      pltpu.VMEM((RC, H), jnp.float32),
      pltpu.SemaphoreType.DMA,
      pltpu.SemaphoreType.DMA,
      pltpu.SemaphoreType.DMA,
      pltpu.SemaphoreType.DMA,
      pltpu.SemaphoreType.DMA,
      pltpu.SemaphoreType.DMA,
      pltpu.SemaphoreType.DMA,
  ]

  def body(packed_h, u_h, p0_h, p1_h,
           acc, pbuf, rbuf, gbuf, zbuf, semz, semi0, semi1, semg0, semg1,
           sems0, sems1):
    c = lax.axis_index("c")
    s = lax.axis_index("s")
    semi = [semi0, semi1]
    semg = [semg0, semg1]
    sems = [sems0, sems1]
    per = jnp.where(c == 0, per0, per1)
    my0 = jnp.where(c == 0, 0, e2) + s * per
    cnt = jnp.where(c == 0, e2, etot - e2)
    mycnt = jnp.clip(cnt - s * per, 0, per)
    nch = (mycnt + G - 1) // G

    def zrow(i, _):
      for q in range(NQ):
        zbuf[i, pl.ds(q * 16, 16)] = jnp.zeros((16,), jnp.float32)
      return 0
    lax.fori_loop(0, RC, zrow, 0)

    r0 = s * PT

    def zfire(i, _):
      ri = pl.multiple_of(r0 + i * RC, 8)
      pltpu.async_copy(zbuf, acc.at[pl.ds(ri, RC)], semz)
      return 0
    lax.fori_loop(0, PT // RC, zfire, 0)

    def zdrain(i, _):
      ri = pl.multiple_of(r0 + i * RC, 8)
      pltpu.make_async_copy(zbuf, acc.at[pl.ds(ri, RC)], semz).wait()
      return 0
    lax.fori_loop(0, PT // RC, zdrain, 0)
    plsc.subcore_barrier()

    lanes0 = lax.iota(jnp.int32, 16)

    def pslice(b, part, n=3 * G):
      return pbuf.at[pl.ds(b * 3 * G + part * G, n)]

    def idx_src(i):
      off3 = pl.multiple_of((my0 // G + i) * 3 * G, 8)
      return packed_h.at[pl.ds(off3, 3 * G)]

    def idx_issue(i, b):
      pltpu.async_copy(idx_src(i), pslice(b, 0), semi[b])

    def idx_wait(i, b):
      pltpu.make_async_copy(idx_src(i), pslice(b, 0), semi[b]).wait()

    def gather_issue(b):
      pltpu.async_copy(u_h.at[pslice(b, 0, G)], gbuf.at[b], semg[b])

    def gather_wait(b):
      pltpu.make_async_copy(u_h.at[pslice(b, 0, G)], gbuf.at[b],
                            semg[b]).wait()

    def process(i, b):
      rem = mycnt - i * G

      def grp(j, _):
        vi = pbuf[pl.ds(b * 3 * G + G + j * 16, 16)]
        vv = lax.bitcast_convert_type(vi, jnp.float32)
        vv = jnp.where(j * 16 + lanes0 < rem, vv, 0.0)
        for l in range(16):
          val = vv[l]
          g = j * 16 + l
          for qq in range(NQ):
            sl = pl.ds(qq * 16, 16)
            gbuf[b, g, sl] = gbuf[b, g, sl] * val
        return 0
      lax.fori_loop(0, G // 16, grp, 0)
      for j in range(8):
        sl = pl.ds(j * 16, 16)
        rbuf[b, sl] = pbuf[pl.ds(b * 3 * G + 2 * G + j * 16, 16)]
      pltpu.async_copy(gbuf.at[b], acc.at[rbuf.at[b]], sems[b], add=True)

    def scat_wait(b):
      pltpu.make_async_copy(gbuf.at[b], acc.at[rbuf.at[b]],
                            sems[b]).wait()

    @pl.when(nch > 0)
    def _():
      pltpu.sync_copy(idx_src(0), pslice(0, 0))
      gather_issue(0)

    @pl.when(nch > 1)
    def _():
      idx_issue(1, 1)

    def pair(p, _):
      i0, i1, i2, i3 = 2 * p, 2 * p + 1, 2 * p + 2, 2 * p + 3

      @pl.when(jnp.logical_and(i1 < nch, i1 >= 2))
      def _():
        scat_wait(1)

      @pl.when(i1 < nch)
      def _():
        idx_wait(i1, 1)
        gather_issue(1)

      gather_wait(0)
      process(i0, 0)

      @pl.when(i2 < nch)
      def _():
        idx_issue(i2, 0)
        idx_wait(i2, 0)
        scat_wait(0)
        gather_issue(0)

      @pl.when(i1 < nch)
      def _():
        gather_wait(1)
        process(i1, 1)

      @pl.when(i3 < nch)
      def _():
        idx_issue(i3, 1)
      return 0
    lax.fori_loop(0, (nch + 1) // 2, pair, 0)

    last_par = lax.rem(nch - 1, 2)

    @pl.when(jnp.logical_and(nch >= 1, last_par == 0))
    def _():
      scat_wait(0)

    @pl.when(jnp.logical_and(nch >= 1, last_par == 1))
    def _():
      scat_wait(1)

    prev_par = lax.rem(nch, 2)

    @pl.when(jnp.logical_and(nch >= 2, prev_par == 0))
    def _():
      scat_wait(0)

    @pl.when(jnp.logical_and(nch >= 2, prev_par == 1))
    def _():
      scat_wait(1)
    plsc.subcore_barrier()

    @pl.when(jnp.logical_and(s == 0, c == 0))
    def _():
      pltpu.sync_copy(acc, p0_h)

    @pl.when(jnp.logical_and(s == 0, c == 1))
    def _():
      pltpu.sync_copy(acc, p1_h)

  out_type = [jax.ShapeDtypeStruct((npad, H), jnp.float32)] * 2
  return pl.kernel(body, out_type=out_type, mesh=mesh,
                   scratch_types=scratch, name="cheb_spmm")


def _edge_pad_len(E, e2):
  per0 = -(-(-(-e2 // NTILE)) // G) * G
  per1 = -(-(-(-(E - e2) // NTILE)) // G) * G
  end0 = NTILE * per0 + SUPER * G
  end1 = e2 + NTILE * per1 + SUPER * G
  n = max(end0, end1, E)
  return -(-n // G) * G, per0, per1


@functools.lru_cache(maxsize=None)
def _make_deg(npad, ep, e2, etot):
  mesh = plsc.VectorSubcoreMesh(core_axis_name="c", subcore_axis_name="s")
  PT = npad // NTILE

  scratch = [
      pltpu.VMEM_SHARED((npad,), jnp.float32),
      pltpu.VMEM((G,), jnp.int32),
      pltpu.VMEM((G,), jnp.float32),
      pltpu.VMEM((RC * 16,), jnp.float32),
  ]

  def body(rows_h, d0_h, d1_h, accd, rbuf, obuf, zbuf):
    c = lax.axis_index("c")
    s = lax.axis_index("s")
    my0, mycnt = _edge_split(e2, etot, c, s)
    nch = (mycnt + G - 1) // G

    def zrow(i, _):
      zbuf[pl.ds(i * 16, 16)] = jnp.zeros((16,), jnp.float32)
      return 0
    lax.fori_loop(0, RC, zrow, 0)
    r0 = s * PT

    def zc(i, _):
      ri = pl.multiple_of(r0 + i * RC * 16, 8)
      pltpu.sync_copy(zbuf, accd.at[pl.ds(ri, RC * 16)])
      return 0
    lax.fori_loop(0, PT // (RC * 16), zc, 0)
    plsc.subcore_barrier()

    lanes0 = lax.iota(jnp.int32, 16)

    def chunk(i, _):
      off = pl.multiple_of(my0 + i * G, 8)
      pltpu.sync_copy(rows_h.at[pl.ds(off, G)], rbuf)
      rem = mycnt - i * G
      for j in range(G // 16):
        ones = jnp.where(j * 16 + lanes0 < rem, 1.0, 0.0)
        obuf[pl.ds(j * 16, 16)] = ones
      pltpu.sync_copy(obuf, accd.at[rbuf], add=True)
      return 0
    lax.fori_loop(0, nch, chunk, 0)
    plsc.subcore_barrier()

    @pl.when(jnp.logical_and(s == 0, c == 0))
    def _():
      pltpu.sync_copy(accd, d0_h)

    @pl.when(jnp.logical_and(s == 0, c == 1))
    def _():
      pltpu.sync_copy(accd, d1_h)

  out_type = [jax.ShapeDtypeStruct((npad,), jnp.float32)] * 2
  return pl.kernel(body, out_type=out_type, mesh=mesh,
                   scratch_types=scratch, name="deg_count")


def _degmax_tc(d0, d1):
  def body(a_ref, b_ref, o_ref):
    m = jnp.max(a_ref[...] + b_ref[...])
    o_ref[...] = jnp.full((8, 128), m, jnp.float32)
  return pl.pallas_call(
      body, out_shape=jax.ShapeDtypeStruct((8, 128), jnp.float32))(d0, d1)


@functools.lru_cache(maxsize=None)
def _make_merge(npad, H, aux):
  grid = npad // BR

  def body(*refs):
    if aux:
      coef, p0, p1, u, p, o, xa, ta, t_out, o_out = refs
    else:
      coef, p0, p1, u, p, o, t_out, o_out = refs
    a_, b_, d_ = coef[0], coef[1], coef[2]
    f0, f1, f2 = coef[3], coef[4], coef[5]
    uv = u[...]
    t = a_ * (p0[...] + p1[...]) + b_ * uv + d_ * p[...]
    oo = f0 * o[...] + f1 * uv + f2 * t
    if aux:
      oo = oo + coef[6] * xa[...] + coef[7] * ta[...]
    t_out[...] = t
    o_out[...] = oo

  narr = 7 if aux else 5
  bspec = pl.BlockSpec((BR, H), lambda i: (i, 0))
  return pl.pallas_call(
      body,
      grid=(grid,),
      in_specs=[pl.BlockSpec(memory_space=pltpu.SMEM)] + [bspec] * narr,
      out_specs=[bspec, bspec],
      out_shape=[jax.ShapeDtypeStruct((npad, H), jnp.float32)] * 2,
  )


def kernel(x, edge_index, edge_vals, poly_logits, hp_alpha):
  N, H = x.shape
  E = edge_index.shape[1]
  K = poly_logits.shape[0] - 1
  npad = -(-N // BR) * BR

  rows = edge_index[0]
  cols = edge_index[1]
  e2 = -(-(-(-E // 2)) // G) * G
  nbg, per0, per1 = _edge_pad_len(E, e2)
  zpad = jnp.zeros((nbg - E,), jnp.int32)
  rows_p = jnp.concatenate([rows, zpad])
  cols_p = jnp.concatenate([cols, zpad])
  vbits = lax.bitcast_convert_type(edge_vals, jnp.int32)
  vals_p = jnp.concatenate([vbits, zpad])
  nb = nbg // G
  packed = jnp.stack([cols_p.reshape(nb, G), vals_p.reshape(nb, G),
                      rows_p.reshape(nb, G)], axis=1).reshape(-1)
  ep = nbg

  xp = jnp.concatenate([x, jnp.zeros((npad - N, H), jnp.float32)])

  deg = _make_deg(npad, ep, e2, E)
  d0, d1 = deg(rows_p)
  dm = _degmax_tc(d0.reshape(npad // 128, 128), d1.reshape(npad // 128, 128))
  lam = 2.0 * dm[0, 0]
  c2 = 2.0 / lam

  w = jax.nn.softmax(poly_logits)
  alpha = hp_alpha.astype(jnp.float32)

  def mk(a, b, d, f0, f1, f2, f3=0.0, f4=0.0):
    return jnp.stack([jnp.asarray(v, jnp.float32) * jnp.ones((), jnp.float32)
                      for v in (a, b, d, f0, f1, f2, f3, f4)])

  spmm = _make_spmm(npad, H, e2, E, per0, per1)
  merge = _make_merge(npad, H, False)
  merge_aux = _make_merge(npad, H, True)

  p0, p1 = spmm(packed, xp)
  t1, out = merge(mk(c2, -1.0, 0.0, 0.0, w[0], w[1]), p0, p1, xp, xp, xp)
  tprev, tcur = xp, t1
  for k in range(1, K):
    p0, p1 = spmm(packed, tcur)
    if k < K - 1:
      coef = mk(2.0 * c2, -2.0, -1.0, 1.0, 0.0, w[k + 1])
      tnext, out = merge(coef, p0, p1, tcur, tprev, out)
    else:
      coef = mk(2.0 * c2, -2.0, -1.0, -1.0, 0.0, -w[k + 1],
                alpha * 0.5, -alpha * 0.5)
      tnext, out = merge_aux(coef, p0, p1, tcur, tprev, out, xp, t1)
    tprev, tcur = tcur, tnext
  return out[:N]
